# baseline (speedup 1.0000x reference)
import functools

import jax
import jax.numpy as jnp
from jax.experimental import pallas as pl
from jax.experimental.pallas import tpu as pltpu


def _dense_body(x_ref, w_ref, o_ref):
    o_ref[...] = jnp.dot(
        x_ref[...].astype(jnp.bfloat16),
        w_ref[...].astype(jnp.bfloat16),
        preferred_element_type=jnp.float32,
    )


def _dense(x, w):
    n = x.shape[0]
    h = w.shape[1]
    return pl.pallas_call(
        _dense_body,
        out_shape=jax.ShapeDtypeStruct((n, h), jnp.float32),
    )(x, w)


def _agg_body(adj_ref, s_ref, b_ref, o_ref, *, relu):
    acc = jnp.dot(
        adj_ref[...].astype(jnp.bfloat16),
        s_ref[...].astype(jnp.bfloat16),
        preferred_element_type=jnp.float32,
    ) + b_ref[...]
    if relu:
        acc = jnp.maximum(acc, 0.0)
    o_ref[...] = acc


def _aggregate(adj, s, b, relu, bi):
    n = adj.shape[0]
    h = s.shape[1]
    return pl.pallas_call(
        functools.partial(_agg_body, relu=relu),
        grid=(n // bi,),
        in_specs=[
            pl.BlockSpec((bi, n), lambda i: (i, 0)),
            pl.BlockSpec((n, h), lambda i: (0, 0)),
            pl.BlockSpec((1, h), lambda i: (0, 0)),
        ],
        out_specs=pl.BlockSpec((bi, h), lambda i: (i, 0)),
        out_shape=jax.ShapeDtypeStruct((n, h), jnp.float32),
        compiler_params=pltpu.CompilerParams(
            dimension_semantics=("arbitrary",)
        ),
    )(adj, s, b)


def kernel(x, adj, W1, b1, W2, b2):
    s1 = _dense(x, W1)
    x1 = _aggregate(adj, s1, b1.reshape(1, -1), True, 400)
    s2 = _dense(x1, W2)
    x2 = _aggregate(adj, s2, b2.reshape(1, -1), False, 400)
    return (x1, x2)
